# baseline (speedup 1.0000x reference)
import functools

import jax
import jax.numpy as jnp
from jax import lax
from jax.experimental import pallas as pl
from jax.experimental.pallas import tpu as pltpu
from jax.experimental.pallas import tpu_sc as plsc

B = 16384
D = 16


def _make_sc_gather():
    info = plsc.get_sparse_core_info()
    nc, ns = info.num_cores, info.num_subcores
    nw = nc * ns
    bpw = B // nw

    mesh = plsc.VectorSubcoreMesh(core_axis_name="c", subcore_axis_name="s")

    @functools.partial(
        pl.kernel,
        mesh=mesh,
        compiler_params=pltpu.CompilerParams(use_tc_tiling_on_sc=False),
        out_type=[
            jax.ShapeDtypeStruct((B, D), jnp.float32),
            jax.ShapeDtypeStruct((B, D), jnp.float32),
            jax.ShapeDtypeStruct((B, D), jnp.float32),
            jax.ShapeDtypeStruct((8, D), jnp.float32),
        ],
        scratch_types=[
            pltpu.VMEM((bpw,), jnp.int32),
            pltpu.VMEM((bpw,), jnp.int32),
            pltpu.VMEM((bpw,), jnp.int32),
            pltpu.VMEM((8,), jnp.int32),
            pltpu.VMEM((bpw, D), jnp.float32),
            pltpu.VMEM((bpw, D), jnp.float32),
            pltpu.VMEM((bpw, D), jnp.float32),
            pltpu.VMEM((8, D), jnp.float32),
            pltpu.SemaphoreType.DMA,
            pltpu.SemaphoreType.DMA,
            pltpu.SemaphoreType.DMA,
            pltpu.SemaphoreType.DMA,
        ],
    )
    def sc_gather(
        user_hbm, item_hbm, dir_hbm, g8_hbm,
        ut_hbm, it_hbm, dt_hbm, gt_hbm,
        out_u, out_i, out_d, out_g,
        idx_u, idx_i, idx_d, idx_g,
        rows_u, rows_i, rows_d, rows_g,
        sem_u, sem_i, sem_d, sem_g,
    ):
        wid = lax.axis_index("s") * nc + lax.axis_index("c")
        base = wid * bpw
        sl = pl.ds(base, bpw)

        pltpu.sync_copy(user_hbm.at[sl], idx_u)
        pltpu.sync_copy(item_hbm.at[sl], idx_i)
        pltpu.sync_copy(dir_hbm.at[sl], idx_d)

        cp_u = pltpu.async_copy(ut_hbm.at[idx_u], rows_u, sem_u)
        cp_i = pltpu.async_copy(it_hbm.at[idx_i], rows_i, sem_i)
        cp_d = pltpu.async_copy(dt_hbm.at[idx_d], rows_d, sem_d)

        @pl.when(wid == 0)
        def _genre():
            pltpu.sync_copy(g8_hbm, idx_g)
            pltpu.async_copy(gt_hbm.at[idx_g], rows_g, sem_g).wait()
            pltpu.sync_copy(rows_g, out_g)

        cp_u.wait()
        pltpu.sync_copy(rows_u, out_u.at[sl])
        cp_i.wait()
        pltpu.sync_copy(rows_i, out_i.at[sl])
        cp_d.wait()
        pltpu.sync_copy(rows_d, out_d.at[sl])

    return sc_gather


_sc_gather = _make_sc_gather()


def _tc_body(eu_ref, ei_ref, ed_ref, year_ref, g_ref, fcw_ref, bias_ref,
             w1_ref, b1_ref, w2_ref, b2_ref, w3_ref, b3_ref, y_ref):
    eu = eu_ref[...]
    ei = ei_ref[...]
    ed = ed_ref[...]
    g = g_ref[0:1, :]
    yr = year_ref[...]

    v = eu + ei + ed + g
    s = jnp.sum(v, axis=1, keepdims=True)
    ssq = jnp.sum(v * v, axis=1, keepdims=True)
    second = 0.5 * (s * s - ssq)

    fcw = fcw_ref[...]
    fm = (
        jnp.dot(eu, fcw[0:D], preferred_element_type=jnp.float32)
        + jnp.dot(ei, fcw[D:2 * D], preferred_element_type=jnp.float32)
        + jnp.dot(ed, fcw[3 * D:4 * D], preferred_element_type=jnp.float32)
        + jnp.dot(g, fcw[2 * D:3 * D], preferred_element_type=jnp.float32)
        + bias_ref[...]
        + second
        + yr
    )

    w1 = w1_ref[...]
    p = (
        jnp.dot(eu, w1[0:D], preferred_element_type=jnp.float32)
        + jnp.dot(ei, w1[D:2 * D], preferred_element_type=jnp.float32)
        + jnp.dot(ed, w1[2 * D:3 * D], preferred_element_type=jnp.float32)
        + jnp.dot(g, w1[3 * D:4 * D], preferred_element_type=jnp.float32)
        + yr * w1[4 * D:4 * D + 1]
        + b1_ref[...]
    )
    h1 = jnp.maximum(p, 0.0)
    h2 = jnp.maximum(
        jnp.dot(h1, w2_ref[...], preferred_element_type=jnp.float32)
        + b2_ref[...], 0.0)
    mlp = jnp.dot(h2, w3_ref[...], preferred_element_type=jnp.float32) \
        + b3_ref[...]

    y_ref[...] = jax.nn.sigmoid((fm + mlp)[:, 0])


def _tc_dense(eu, ei, ed, year, grow, fc_w, bias2, w1, b1_2, w2, b2_2, w3,
              b3_2, blk):
    grid = (B // blk,)
    full = lambda shape: pl.BlockSpec(shape, lambda i: (0, 0))
    return pl.pallas_call(
        _tc_body,
        grid=grid,
        in_specs=[
            pl.BlockSpec((blk, D), lambda i: (i, 0)),
            pl.BlockSpec((blk, D), lambda i: (i, 0)),
            pl.BlockSpec((blk, D), lambda i: (i, 0)),
            pl.BlockSpec((blk, 1), lambda i: (i, 0)),
            full((8, D)),
            full((4 * D, 1)),
            full((1, 1)),
            full((4 * D + 1, 64)),
            full((1, 64)),
            full((64, 32)),
            full((1, 32)),
            full((32, 1)),
            full((1, 1)),
        ],
        out_specs=pl.BlockSpec((blk,), lambda i: (i,)),
        out_shape=jax.ShapeDtypeStruct((B,), jnp.float32),
    )(eu, ei, ed, year, grow, fc_w, bias2, w1, b1_2, w2, b2_2, w3, b3_2)


def kernel(user, item, genre, director, year, user_table, item_table,
           genre_table, director_table, fc_w, bias, w1, b1, w2, b2, w3, b3):
    user = user.astype(jnp.int32)
    item = item.astype(jnp.int32)
    director = director.astype(jnp.int32)
    g8 = jnp.broadcast_to(genre.reshape(-1)[0:1].astype(jnp.int32), (8,))

    eu, ei, ed, grow = _sc_gather(
        user, item, director, g8,
        user_table, item_table, director_table, genre_table)

    return _tc_dense(
        eu, ei, ed, year, grow,
        fc_w, bias.reshape(1, 1), w1, b1.reshape(1, -1), w2,
        b2.reshape(1, -1), w3, b3.reshape(1, 1), blk=2048)
